# baseline (device time: 744315 ns/iter reference)
import jax
import jax.numpy as jnp
from jax import lax
from jax.experimental import pallas as pl
from jax.experimental.pallas import tpu as pltpu

N = 32


def kernel(x, Win0, Wout0, Win1, Wout1, Win2, Wout2):
    B, D = x.shape
    M = N * B

    def body(x_ref, win0, wout0, win1, wout1, win2, wout2, out_ref,
             xfull, part, rs_buf,
             xag_send, xag_recv, rs_send, rs_recv, ag_send, ag_recv):
        me = lax.axis_index("i")
        right = lax.rem(me + 1, N)
        left = lax.rem(me + N - 1, N)

        barrier = pltpu.get_barrier_semaphore()
        pl.semaphore_signal(barrier, inc=1, device_id=(left,),
                            device_id_type=pl.DeviceIdType.MESH)
        pl.semaphore_signal(barrier, inc=1, device_id=(right,),
                            device_id_type=pl.DeviceIdType.MESH)
        pl.semaphore_wait(barrier, 2)

        def ring_ag(target, off, send_sems, recv_sems):
            sends = []
            for h in range(N - 1):
                cs = lax.rem(me + off + 2 * N - h, N)
                cr = lax.rem(me + off + 2 * N - h - 1, N)
                send = pltpu.make_async_remote_copy(
                    src_ref=target.at[pl.ds(cs * B, B), :],
                    dst_ref=target.at[pl.ds(cs * B, B), :],
                    send_sem=send_sems.at[h], recv_sem=recv_sems.at[h],
                    device_id=(right,), device_id_type=pl.DeviceIdType.MESH)
                send.start()
                sends.append(send)
                recv = pltpu.make_async_remote_copy(
                    src_ref=target.at[pl.ds(cr * B, B), :],
                    dst_ref=target.at[pl.ds(cr * B, B), :],
                    send_sem=send_sems.at[h], recv_sem=recv_sems.at[h],
                    device_id=(right,), device_id_type=pl.DeviceIdType.MESH)
                recv.wait_recv()
            for s in sends:
                s.wait_send()

        xfull[pl.ds(me * B, B), :] = x_ref[:, :]
        ring_ag(xfull, 0, xag_send, xag_recv)

        layers = ((win0, wout0), (win1, wout1), (win2, wout2))
        for l, (win, wout) in enumerate(layers):
            hmat = jnp.maximum(
                jnp.dot(xfull[:, :], win[:, :],
                        preferred_element_type=jnp.float32), 0.0)
            part[:, :] = jnp.dot(hmat, wout[:, :],
                                 preferred_element_type=jnp.float32)

            rs_sends = []
            for h in range(N - 1):
                cs = lax.rem(me + 2 * N - h, N)
                cr = lax.rem(me + 2 * N - h - 1, N)
                rdma = pltpu.make_async_remote_copy(
                    src_ref=part.at[pl.ds(cs * B, B), :],
                    dst_ref=rs_buf.at[h],
                    send_sem=rs_send.at[h], recv_sem=rs_recv.at[h],
                    device_id=(right,), device_id_type=pl.DeviceIdType.MESH)
                rdma.start()
                rs_sends.append(rdma)
                rdma.wait_recv()
                part[pl.ds(cr * B, B), :] = (
                    part[pl.ds(cr * B, B), :] + rs_buf[h, :, :])
            for s in rs_sends:
                s.wait_send()

            target = out_ref if l == 2 else xfull
            c_red = lax.rem(me + 1, N)
            target[pl.ds(c_red * B, B), :] = part[pl.ds(c_red * B, B), :]
            ring_ag(target, 1, ag_send, ag_recv)

    return pl.pallas_call(
        body,
        out_shape=jax.ShapeDtypeStruct((M, D), jnp.float32),
        in_specs=[pl.BlockSpec(memory_space=pltpu.VMEM)] * 7,
        out_specs=pl.BlockSpec(memory_space=pltpu.VMEM),
        scratch_shapes=[
            pltpu.VMEM((M, D), jnp.float32),
            pltpu.VMEM((M, D), jnp.float32),
            pltpu.VMEM((N - 1, B, D), jnp.float32),
            pltpu.SemaphoreType.DMA((N - 1,)),
            pltpu.SemaphoreType.DMA((N - 1,)),
            pltpu.SemaphoreType.DMA((N - 1,)),
            pltpu.SemaphoreType.DMA((N - 1,)),
            pltpu.SemaphoreType.DMA((N - 1,)),
            pltpu.SemaphoreType.DMA((N - 1,)),
        ],
        compiler_params=pltpu.CompilerParams(collective_id=0),
    )(x, Win0, Wout0, Win1, Wout1, Win2, Wout2)


# device time: 422714 ns/iter; 1.7608x vs baseline; 1.7608x over previous
import jax
import jax.numpy as jnp
from jax import lax
from jax.experimental import pallas as pl
from jax.experimental.pallas import tpu as pltpu

N = 32
MASKS_RS = (1, 8, 2, 4, 16)
MASKS_AG = (16, 4, 2, 8, 1)


def _span(masks):
    out = [0]
    for m in masks:
        out = out + [o ^ m for o in out]
    return out


def kernel(x, Win0, Wout0, Win1, Wout1, Win2, Wout2):
    B, D = x.shape
    M = N * B

    def body(x_ref, win0, wout0, win1, wout1, win2, wout2, out_ref,
             xfull, part, stage,
             xag_send, xag_recv, rs_send, rs_recv, ag_send, ag_recv):
        me = lax.axis_index("i")

        barrier = pltpu.get_barrier_semaphore()
        for m in (1, 2, 4, 8, 16):
            pl.semaphore_signal(barrier, inc=1,
                                device_id=(lax.bitwise_xor(me, m),),
                                device_id_type=pl.DeviceIdType.MESH)
        pl.semaphore_wait(barrier, 5)

        def butterfly_ag(target, send_sems, recv_sems):
            slot = 0
            sends = []
            for p, mask in enumerate(MASKS_AG):
                partner = lax.bitwise_xor(me, mask)
                recvs = []
                for c in _span(MASKS_AG[:p]):
                    b_mine = lax.bitwise_xor(me, c)
                    b_their = lax.bitwise_xor(me, c ^ mask)
                    snd = pltpu.make_async_remote_copy(
                        src_ref=target.at[pl.ds(b_mine * B, B), :],
                        dst_ref=target.at[pl.ds(b_mine * B, B), :],
                        send_sem=send_sems.at[slot],
                        recv_sem=recv_sems.at[slot],
                        device_id=(partner,),
                        device_id_type=pl.DeviceIdType.MESH)
                    snd.start()
                    sends.append(snd)
                    rcv = pltpu.make_async_remote_copy(
                        src_ref=target.at[pl.ds(b_their * B, B), :],
                        dst_ref=target.at[pl.ds(b_their * B, B), :],
                        send_sem=send_sems.at[slot],
                        recv_sem=recv_sems.at[slot],
                        device_id=(partner,),
                        device_id_type=pl.DeviceIdType.MESH)
                    recvs.append(rcv)
                    slot += 1
                for rcv in recvs:
                    rcv.wait_recv()
            for s in sends:
                s.wait_send()

        xfull[pl.ds(me * B, B), :] = x_ref[:, :]
        butterfly_ag(xfull, xag_send, xag_recv)

        layers = ((win0, wout0), (win1, wout1), (win2, wout2))
        for l, (win, wout) in enumerate(layers):
            hmat = jnp.maximum(
                jnp.dot(xfull[:, :], win[:, :],
                        preferred_element_type=jnp.float32), 0.0)
            part[:, :] = jnp.dot(hmat, wout[:, :],
                                 preferred_element_type=jnp.float32)

            slot = 0
            sends = []
            for r, mask in enumerate(MASKS_RS):
                partner = lax.bitwise_xor(me, mask)
                recvs = []
                for c in _span(MASKS_RS[r + 1:]):
                    b_send = lax.bitwise_xor(me, c ^ mask)
                    b_keep = lax.bitwise_xor(me, c)
                    snd = pltpu.make_async_remote_copy(
                        src_ref=part.at[pl.ds(b_send * B, B), :],
                        dst_ref=stage.at[pl.ds(slot * B, B), :],
                        send_sem=rs_send.at[slot],
                        recv_sem=rs_recv.at[slot],
                        device_id=(partner,),
                        device_id_type=pl.DeviceIdType.MESH)
                    snd.start()
                    sends.append(snd)
                    recvs.append((snd, b_keep, slot))
                    slot += 1
                for rcv, b_keep, sl in recvs:
                    rcv.wait_recv()
                    part[pl.ds(b_keep * B, B), :] = (
                        part[pl.ds(b_keep * B, B), :]
                        + stage[pl.ds(sl * B, B), :])
            for s in sends:
                s.wait_send()

            target = out_ref if l == 2 else xfull
            target[pl.ds(me * B, B), :] = part[pl.ds(me * B, B), :]
            butterfly_ag(target, ag_send, ag_recv)

    return pl.pallas_call(
        body,
        out_shape=jax.ShapeDtypeStruct((M, D), jnp.float32),
        in_specs=[pl.BlockSpec(memory_space=pltpu.VMEM)] * 7,
        out_specs=pl.BlockSpec(memory_space=pltpu.VMEM),
        scratch_shapes=[
            pltpu.VMEM((M, D), jnp.float32),
            pltpu.VMEM((M, D), jnp.float32),
            pltpu.VMEM(((N - 1) * B, D), jnp.float32),
            pltpu.SemaphoreType.DMA((N - 1,)),
            pltpu.SemaphoreType.DMA((N - 1,)),
            pltpu.SemaphoreType.DMA((N - 1,)),
            pltpu.SemaphoreType.DMA((N - 1,)),
            pltpu.SemaphoreType.DMA((N - 1,)),
            pltpu.SemaphoreType.DMA((N - 1,)),
        ],
        compiler_params=pltpu.CompilerParams(collective_id=0),
    )(x, Win0, Wout0, Win1, Wout1, Win2, Wout2)
